# baseline (speedup 1.0000x reference)
import functools

import jax
import jax.numpy as jnp
from jax import lax
from jax.experimental import pallas as pl
from jax.experimental.pallas import tpu as pltpu
from jax.experimental.pallas import tpu_sc as plsc

_N = 10000
_D = 128
_E = 320000
_NC = 2
_NS = 16
_NW = _NC * _NS
_G = 64
_NCH = 160
_NCH0 = 64
_NCH1 = 256
_EPW = _NCH * _G
_EPAD = _NW * _EPW
_ACC_ROWS = 10240
_ROWS_T = _ACC_ROWS // _NS
_DEG_T = _ACC_ROWS // _NS
_BM = 640
_NBUF = 4
_CB = 32

@functools.lru_cache(maxsize=None)
def _sc_kernels():
    mesh = plsc.VectorSubcoreMesh(
        core_axis_name="c", subcore_axis_name="s",
        num_cores=_NC, num_subcores=_NS,
    )

    @functools.partial(
        pl.kernel,
        out_type=[
            jax.ShapeDtypeStruct((_ACC_ROWS,), jnp.float32),
            jax.ShapeDtypeStruct((_ACC_ROWS,), jnp.float32),
        ],
        mesh=mesh,
        scratch_types=[
            pltpu.VMEM((_NCH, _G), jnp.int32),
            pltpu.VMEM((_G,), jnp.float32),
            pltpu.VMEM((_DEG_T,), jnp.float32),
            pltpu.VMEM_SHARED((_ACC_ROWS,), jnp.float32),
        ],
    )
    def deg_kernel(dst3, out_a, out_b, didx, ones, zbuf, deg_sh):
        c = lax.axis_index("c")
        s = lax.axis_index("s")
        wid = c * _NS + s
        for i in range(_DEG_T // 16):
            zbuf[pl.ds(i * 16, 16)] = jnp.zeros((16,), jnp.float32)
        for i in range(_G // 16):
            ones[pl.ds(i * 16, 16)] = jnp.ones((16,), jnp.float32)
        pltpu.sync_copy(zbuf, deg_sh.at[pl.ds(s * _DEG_T, _DEG_T)])
        pltpu.sync_copy(dst3.at[wid], didx)
        plsc.subcore_barrier()

        def body(ch, carry):
            pltpu.sync_copy(ones, deg_sh.at[didx.at[ch]], add=True)
            return carry

        lax.fori_loop(0, _NCH, body, 0)
        plsc.subcore_barrier()

        @pl.when(c == 0)
        def _():
            pltpu.sync_copy(
                deg_sh.at[pl.ds(s * _DEG_T, _DEG_T)],
                out_a.at[pl.ds(s * _DEG_T, _DEG_T)],
            )

        @pl.when(c == 1)
        def _():
            pltpu.sync_copy(
                deg_sh.at[pl.ds(s * _DEG_T, _DEG_T)],
                out_b.at[pl.ds(s * _DEG_T, _DEG_T)],
            )

    @functools.partial(
        pl.kernel,
        out_type=jax.ShapeDtypeStruct((_NC, _ACC_ROWS, _D), jnp.float32),
        mesh=mesh,
        scratch_types=[
            pltpu.VMEM((_CB, _G), jnp.int32),
            pltpu.VMEM((_CB, _G), jnp.int32),
            pltpu.VMEM((_NBUF, _G, _D), jnp.float32),
            pltpu.VMEM_SHARED((_ACC_ROWS, _D), jnp.float32),
        ] + [pltpu.SemaphoreType.DMA] * _NBUF,
    )
    def spmm_kernel(y, src3, dst3, out, sidx, didx, rows, acc, *sems):
        c = lax.axis_index("c")
        s = lax.axis_index("s")
        wid = c * _NS + s
        rb = s * _ROWS_T
        pltpu.sync_copy(y.at[pl.ds(rb, _ROWS_T)], acc.at[pl.ds(rb, _ROWS_T)])
        plsc.subcore_barrier()
        npair = _CB // _NBUF
        nbatch = jnp.where(c == 0, _NCH0 // _CB, _NCH1 // _CB)

        def batch_body(bt, carry):
            cb0 = bt * _CB
            pltpu.sync_copy(src3.at[wid, pl.ds(cb0, _CB)], sidx)
            pltpu.sync_copy(dst3.at[wid, pl.ds(cb0, _CB)], didx)
            for b in range(_NBUF):
                pltpu.async_copy(y.at[sidx.at[b]], rows.at[b], sems[b])

            def chunk_body(t, carry2):
                base = t * _NBUF
                for b in range(_NBUF):
                    i = base + b
                    pltpu.make_async_copy(
                        y.at[sidx.at[i]], rows.at[b], sems[b]
                    ).wait()
                    pltpu.sync_copy(rows.at[b], acc.at[didx.at[i]], add=True)

                    @pl.when(t < npair - 1)
                    def _(b=b, i=i):
                        pltpu.async_copy(
                            y.at[sidx.at[i + _NBUF]], rows.at[b], sems[b]
                        )

                return carry2

            lax.fori_loop(0, npair, chunk_body, 0)
            return carry

        lax.fori_loop(0, nbatch, batch_body, 0)
        plsc.subcore_barrier()
        pltpu.sync_copy(acc.at[pl.ds(rb, _ROWS_T)], out.at[c, pl.ds(rb, _ROWS_T)])

    return deg_kernel, spmm_kernel


def _mmA_body(d0, d1, x, w, y, dis):
    disv = lax.rsqrt(d0[...] + d1[...] + 1.0)
    y[...] = disv * jnp.dot(x[...], w[...], preferred_element_type=jnp.float32)
    dis[...] = disv


def _mmA(d0, d1, x, w):
    return pl.pallas_call(
        _mmA_body,
        grid=(_ACC_ROWS // _BM,),
        in_specs=[
            pl.BlockSpec((_BM, 1), lambda i: (i, 0)),
            pl.BlockSpec((_BM, 1), lambda i: (i, 0)),
            pl.BlockSpec((_BM, _D), lambda i: (i, 0)),
            pl.BlockSpec((_D, _D), lambda i: (0, 0)),
        ],
        out_specs=[
            pl.BlockSpec((_BM, _D), lambda i: (i, 0)),
            pl.BlockSpec((_BM, 1), lambda i: (i, 0)),
        ],
        out_shape=[
            jax.ShapeDtypeStruct((_ACC_ROWS, _D), jnp.float32),
            jax.ShapeDtypeStruct((_ACC_ROWS, 1), jnp.float32),
        ],
    )(d0, d1, x, w)


def _mmB_body(p, y1, dis, w, b, y2):
    pv = p[...]
    a = pv[0] + pv[1] - y1[...]
    h = jnp.maximum(dis[...] * a + b[...], 0.0)
    y2[...] = dis[...] * jnp.dot(h, w[...], preferred_element_type=jnp.float32)


def _mmB(p, y1, dis, w, b):
    return pl.pallas_call(
        _mmB_body,
        grid=(_ACC_ROWS // _BM,),
        in_specs=[
            pl.BlockSpec((_NC, _BM, _D), lambda i: (0, i, 0)),
            pl.BlockSpec((_BM, _D), lambda i: (i, 0)),
            pl.BlockSpec((_BM, 1), lambda i: (i, 0)),
            pl.BlockSpec((_D, _D), lambda i: (0, 0)),
            pl.BlockSpec((1, _D), lambda i: (0, 0)),
        ],
        out_specs=pl.BlockSpec((_BM, _D), lambda i: (i, 0)),
        out_shape=jax.ShapeDtypeStruct((_ACC_ROWS, _D), jnp.float32),
    )(p, y1, dis, w, b)


def _mmC_body(q, y2, dis, b, o):
    qv = q[...]
    o[...] = dis[...] * (qv[0] + qv[1] - y2[...]) + b[...]


def _mmC(q, y2, dis, b):
    return pl.pallas_call(
        _mmC_body,
        grid=(_ACC_ROWS // _BM,),
        in_specs=[
            pl.BlockSpec((_NC, _BM, _D), lambda i: (0, i, 0)),
            pl.BlockSpec((_BM, _D), lambda i: (i, 0)),
            pl.BlockSpec((_BM, 1), lambda i: (i, 0)),
            pl.BlockSpec((1, _D), lambda i: (0, 0)),
        ],
        out_specs=pl.BlockSpec((_BM, _D), lambda i: (i, 0)),
        out_shape=jax.ShapeDtypeStruct((_ACC_ROWS, _D), jnp.float32),
    )(q, y2, dis, b)


def _split(v, padval):
    ea = _NS * _NCH0 * _G
    eb_cap = _NS * _NCH1 * _G
    pa = jnp.pad(
        v[:ea].reshape(_NS, _NCH0, _G), ((0, 0), (0, _NCH1 - _NCH0), (0, 0))
    )
    pb = jnp.concatenate(
        [v[ea:], jnp.full((eb_cap - (_E - ea),), padval, jnp.int32)]
    ).reshape(_NS, _NCH1, _G)
    return jnp.concatenate([pa, pb])


def kernel(x, edge_index, W1, b1, W2, b2):
    ei = edge_index.astype(jnp.int32)
    src = ei[0]
    dst = ei[1]
    npad = _EPAD - _E
    srcp = jnp.concatenate([src, jnp.zeros((npad,), jnp.int32)]).reshape(
        _NW, _NCH, _G
    )
    dstp = jnp.concatenate([dst, jnp.full((npad,), _N, jnp.int32)]).reshape(
        _NW, _NCH, _G
    )
    srcq = _split(src, 0)
    dstq = _split(dst, _N)
    deg_kernel, spmm_kernel = _sc_kernels()
    dega, degb = deg_kernel(dstp)
    xp = jnp.concatenate(
        [x, jnp.zeros((_ACC_ROWS - _N, _D), jnp.float32)]
    )
    y1, dis = _mmA(dega[:, None], degb[:, None], xp, W1)
    p = spmm_kernel(y1, srcq, dstq)
    y2 = _mmB(p, y1, dis, W2, b1.reshape(1, _D))
    q = spmm_kernel(y2, srcq, dstq)
    out = _mmC(q, y2, dis, b2.reshape(1, _D))
    return out[:_N]


# speedup vs baseline: 1.0294x; 1.0294x over previous
import functools

import jax
import jax.numpy as jnp
from jax import lax
from jax.experimental import pallas as pl
from jax.experimental.pallas import tpu as pltpu
from jax.experimental.pallas import tpu_sc as plsc

_N = 10000
_D = 128
_E = 320000
_NC = 2
_NS = 16
_NW = _NC * _NS
_G = 64
_NCH = 160
_EPW = _NCH * _G
_EPAD = _NW * _EPW
_ACC_ROWS = 10240
_ROWS_T = _ACC_ROWS // _NS
_DEG_T = _ACC_ROWS // _NS
_BM = 640
_NBUF = 4
_CB = 32

@functools.lru_cache(maxsize=None)
def _sc_kernels():
    mesh = plsc.VectorSubcoreMesh(
        core_axis_name="c", subcore_axis_name="s",
        num_cores=_NC, num_subcores=_NS,
    )

    @functools.partial(
        pl.kernel,
        out_type=[
            jax.ShapeDtypeStruct((_ACC_ROWS,), jnp.float32),
            jax.ShapeDtypeStruct((_ACC_ROWS,), jnp.float32),
        ],
        mesh=mesh,
        scratch_types=[
            pltpu.VMEM((_NCH, _G), jnp.int32),
            pltpu.VMEM((_G,), jnp.float32),
            pltpu.VMEM((_DEG_T,), jnp.float32),
            pltpu.VMEM_SHARED((_ACC_ROWS,), jnp.float32),
        ],
    )
    def deg_kernel(dst3, out_a, out_b, didx, ones, zbuf, deg_sh):
        c = lax.axis_index("c")
        s = lax.axis_index("s")
        wid = c * _NS + s
        for i in range(_DEG_T // 16):
            zbuf[pl.ds(i * 16, 16)] = jnp.zeros((16,), jnp.float32)
        for i in range(_G // 16):
            ones[pl.ds(i * 16, 16)] = jnp.ones((16,), jnp.float32)
        pltpu.sync_copy(zbuf, deg_sh.at[pl.ds(s * _DEG_T, _DEG_T)])
        pltpu.sync_copy(dst3.at[wid], didx)
        plsc.subcore_barrier()

        def body(ch, carry):
            pltpu.sync_copy(ones, deg_sh.at[didx.at[ch]], add=True)
            return carry

        lax.fori_loop(0, _NCH, body, 0)
        plsc.subcore_barrier()

        @pl.when(c == 0)
        def _():
            pltpu.sync_copy(
                deg_sh.at[pl.ds(s * _DEG_T, _DEG_T)],
                out_a.at[pl.ds(s * _DEG_T, _DEG_T)],
            )

        @pl.when(c == 1)
        def _():
            pltpu.sync_copy(
                deg_sh.at[pl.ds(s * _DEG_T, _DEG_T)],
                out_b.at[pl.ds(s * _DEG_T, _DEG_T)],
            )

    @functools.partial(
        pl.kernel,
        out_type=jax.ShapeDtypeStruct((_NC, _ACC_ROWS, _D), jnp.float32),
        mesh=mesh,
        scratch_types=[
            pltpu.VMEM((_CB, _G), jnp.int32),
            pltpu.VMEM((_CB, _G), jnp.int32),
            pltpu.VMEM((_NBUF, _G, _D), jnp.float32),
            pltpu.VMEM_SHARED((_ACC_ROWS, _D), jnp.float32),
        ] + [pltpu.SemaphoreType.DMA] * _NBUF,
    )
    def spmm_kernel(y, src3, dst3, out, sidx, didx, rows, acc, *sems):
        c = lax.axis_index("c")
        s = lax.axis_index("s")
        wid = c * _NS + s
        rb = s * _ROWS_T
        pltpu.sync_copy(y.at[pl.ds(rb, _ROWS_T)], acc.at[pl.ds(rb, _ROWS_T)])
        plsc.subcore_barrier()
        npair = _CB // _NBUF

        def batch_body(bt, carry):
            cb0 = bt * _CB
            pltpu.sync_copy(src3.at[wid, pl.ds(cb0, _CB)], sidx)
            pltpu.sync_copy(dst3.at[wid, pl.ds(cb0, _CB)], didx)
            for b in range(_NBUF):
                pltpu.async_copy(y.at[sidx.at[b]], rows.at[b], sems[b])

            def chunk_body(t, carry2):
                base = t * _NBUF
                for b in range(_NBUF):
                    i = base + b
                    pltpu.make_async_copy(
                        y.at[sidx.at[i]], rows.at[b], sems[b]
                    ).wait()
                    pltpu.sync_copy(rows.at[b], acc.at[didx.at[i]], add=True)

                    @pl.when(t < npair - 1)
                    def _(b=b, i=i):
                        pltpu.async_copy(
                            y.at[sidx.at[i + _NBUF]], rows.at[b], sems[b]
                        )

                return carry2

            lax.fori_loop(0, npair, chunk_body, 0)
            return carry

        lax.fori_loop(0, _NCH // _CB, batch_body, 0)
        plsc.subcore_barrier()
        pltpu.sync_copy(acc.at[pl.ds(rb, _ROWS_T)], out.at[c, pl.ds(rb, _ROWS_T)])

    return deg_kernel, spmm_kernel


def _mmA_body(d0, d1, x, w, y, dis):
    disv = lax.rsqrt(d0[...] + d1[...] + 1.0)
    y[...] = disv * jnp.dot(x[...], w[...], preferred_element_type=jnp.float32)
    dis[...] = disv


def _mmA(d0, d1, x, w):
    return pl.pallas_call(
        _mmA_body,
        grid=(_ACC_ROWS // _BM,),
        in_specs=[
            pl.BlockSpec((_BM, 1), lambda i: (i, 0)),
            pl.BlockSpec((_BM, 1), lambda i: (i, 0)),
            pl.BlockSpec((_BM, _D), lambda i: (i, 0)),
            pl.BlockSpec((_D, _D), lambda i: (0, 0)),
        ],
        out_specs=[
            pl.BlockSpec((_BM, _D), lambda i: (i, 0)),
            pl.BlockSpec((_BM, 1), lambda i: (i, 0)),
        ],
        out_shape=[
            jax.ShapeDtypeStruct((_ACC_ROWS, _D), jnp.float32),
            jax.ShapeDtypeStruct((_ACC_ROWS, 1), jnp.float32),
        ],
    )(d0, d1, x, w)


def _mmB_body(p, y1, dis, w, b, y2):
    pv = p[...]
    a = pv[0] + pv[1] - y1[...]
    h = jnp.maximum(dis[...] * a + b[...], 0.0)
    y2[...] = dis[...] * jnp.dot(h, w[...], preferred_element_type=jnp.float32)


def _mmB(p, y1, dis, w, b):
    return pl.pallas_call(
        _mmB_body,
        grid=(_ACC_ROWS // _BM,),
        in_specs=[
            pl.BlockSpec((_NC, _BM, _D), lambda i: (0, i, 0)),
            pl.BlockSpec((_BM, _D), lambda i: (i, 0)),
            pl.BlockSpec((_BM, 1), lambda i: (i, 0)),
            pl.BlockSpec((_D, _D), lambda i: (0, 0)),
            pl.BlockSpec((1, _D), lambda i: (0, 0)),
        ],
        out_specs=pl.BlockSpec((_BM, _D), lambda i: (i, 0)),
        out_shape=jax.ShapeDtypeStruct((_ACC_ROWS, _D), jnp.float32),
    )(p, y1, dis, w, b)


def _mmC_body(q, y2, dis, b, o):
    qv = q[...]
    o[...] = dis[...] * (qv[0] + qv[1] - y2[...]) + b[...]


def _mmC(q, y2, dis, b):
    return pl.pallas_call(
        _mmC_body,
        grid=(_ACC_ROWS // _BM,),
        in_specs=[
            pl.BlockSpec((_NC, _BM, _D), lambda i: (0, i, 0)),
            pl.BlockSpec((_BM, _D), lambda i: (i, 0)),
            pl.BlockSpec((_BM, 1), lambda i: (i, 0)),
            pl.BlockSpec((1, _D), lambda i: (0, 0)),
        ],
        out_specs=pl.BlockSpec((_BM, _D), lambda i: (i, 0)),
        out_shape=jax.ShapeDtypeStruct((_ACC_ROWS, _D), jnp.float32),
    )(q, y2, dis, b)


def kernel(x, edge_index, W1, b1, W2, b2):
    ei = edge_index.astype(jnp.int32)
    src = ei[0]
    dst = ei[1]
    npad = _EPAD - _E
    srcp = jnp.concatenate([src, jnp.zeros((npad,), jnp.int32)]).reshape(
        _NW, _NCH, _G
    )
    dstp = jnp.concatenate([dst, jnp.full((npad,), _N, jnp.int32)]).reshape(
        _NW, _NCH, _G
    )
    deg_kernel, spmm_kernel = _sc_kernels()
    dega, degb = deg_kernel(dstp)
    xp = jnp.concatenate(
        [x, jnp.zeros((_ACC_ROWS - _N, _D), jnp.float32)]
    )
    y1, dis = _mmA(dega[:, None], degb[:, None], xp, W1)
    p = spmm_kernel(y1, srcp, dstp)
    y2 = _mmB(p, y1, dis, W2, b1.reshape(1, _D))
    q = spmm_kernel(y2, srcp, dstp)
    out = _mmC(q, y2, dis, b2.reshape(1, _D))
    return out[:_N]


# speedup vs baseline: 1.0303x; 1.0009x over previous
import functools

import jax
import jax.numpy as jnp
from jax import lax
from jax.experimental import pallas as pl
from jax.experimental.pallas import tpu as pltpu
from jax.experimental.pallas import tpu_sc as plsc

_N = 10000
_D = 128
_E = 320000
_NC = 2
_NS = 16
_NW = _NC * _NS
_G = 64
_NCH = 160
_EPW = _NCH * _G
_EPAD = _NW * _EPW
_ACC_ROWS = 10240
_ROWS_T = _ACC_ROWS // _NS
_DEG_T = _ACC_ROWS // _NS
_BM = 640
_NBUF = 4
_CB = 40

@functools.lru_cache(maxsize=None)
def _sc_kernels():
    mesh = plsc.VectorSubcoreMesh(
        core_axis_name="c", subcore_axis_name="s",
        num_cores=_NC, num_subcores=_NS,
    )

    @functools.partial(
        pl.kernel,
        out_type=[
            jax.ShapeDtypeStruct((_ACC_ROWS,), jnp.float32),
            jax.ShapeDtypeStruct((_ACC_ROWS,), jnp.float32),
        ],
        mesh=mesh,
        scratch_types=[
            pltpu.VMEM((_NCH, _G), jnp.int32),
            pltpu.VMEM((_G,), jnp.float32),
            pltpu.VMEM((_DEG_T,), jnp.float32),
            pltpu.VMEM_SHARED((_ACC_ROWS,), jnp.float32),
        ],
    )
    def deg_kernel(dst3, out_a, out_b, didx, ones, zbuf, deg_sh):
        c = lax.axis_index("c")
        s = lax.axis_index("s")
        wid = c * _NS + s
        for i in range(_DEG_T // 16):
            zbuf[pl.ds(i * 16, 16)] = jnp.zeros((16,), jnp.float32)
        for i in range(_G // 16):
            ones[pl.ds(i * 16, 16)] = jnp.ones((16,), jnp.float32)
        pltpu.sync_copy(zbuf, deg_sh.at[pl.ds(s * _DEG_T, _DEG_T)])
        pltpu.sync_copy(dst3.at[wid], didx)
        plsc.subcore_barrier()

        def body(ch, carry):
            pltpu.sync_copy(ones, deg_sh.at[didx.at[ch]], add=True)
            return carry

        lax.fori_loop(0, _NCH, body, 0)
        plsc.subcore_barrier()

        @pl.when(c == 0)
        def _():
            pltpu.sync_copy(
                deg_sh.at[pl.ds(s * _DEG_T, _DEG_T)],
                out_a.at[pl.ds(s * _DEG_T, _DEG_T)],
            )

        @pl.when(c == 1)
        def _():
            pltpu.sync_copy(
                deg_sh.at[pl.ds(s * _DEG_T, _DEG_T)],
                out_b.at[pl.ds(s * _DEG_T, _DEG_T)],
            )

    @functools.partial(
        pl.kernel,
        out_type=jax.ShapeDtypeStruct((_NC, _ACC_ROWS, _D), jnp.float32),
        mesh=mesh,
        scratch_types=[
            pltpu.VMEM((_CB, _G), jnp.int32),
            pltpu.VMEM((_CB, _G), jnp.int32),
            pltpu.VMEM((_NBUF, _G, _D), jnp.float32),
            pltpu.VMEM_SHARED((_ACC_ROWS, _D), jnp.float32),
        ] + [pltpu.SemaphoreType.DMA] * _NBUF,
    )
    def spmm_kernel(y, src3, dst3, out, sidx, didx, rows, acc, *sems):
        c = lax.axis_index("c")
        s = lax.axis_index("s")
        wid = c * _NS + s
        rb = s * _ROWS_T
        pltpu.sync_copy(y.at[pl.ds(rb, _ROWS_T)], acc.at[pl.ds(rb, _ROWS_T)])
        plsc.subcore_barrier()
        npair = _CB // _NBUF

        def batch_body(bt, carry):
            cb0 = bt * _CB
            pltpu.sync_copy(src3.at[wid, pl.ds(cb0, _CB)], sidx)
            pltpu.sync_copy(dst3.at[wid, pl.ds(cb0, _CB)], didx)
            for b in range(_NBUF):
                pltpu.async_copy(y.at[sidx.at[b]], rows.at[b], sems[b])

            def chunk_body(t, carry2):
                base = t * _NBUF
                for b in range(_NBUF):
                    i = base + b
                    pltpu.make_async_copy(
                        y.at[sidx.at[i]], rows.at[b], sems[b]
                    ).wait()
                    pltpu.sync_copy(rows.at[b], acc.at[didx.at[i]], add=True)

                    @pl.when(t < npair - 1)
                    def _(b=b, i=i):
                        pltpu.async_copy(
                            y.at[sidx.at[i + _NBUF]], rows.at[b], sems[b]
                        )

                return carry2

            lax.fori_loop(0, npair, chunk_body, 0)
            return carry

        lax.fori_loop(0, _NCH // _CB, batch_body, 0)
        plsc.subcore_barrier()
        pltpu.sync_copy(acc.at[pl.ds(rb, _ROWS_T)], out.at[c, pl.ds(rb, _ROWS_T)])

    return deg_kernel, spmm_kernel


def _mmA_body(d0, d1, x, w, y, dis):
    disv = lax.rsqrt(d0[...] + d1[...] + 1.0)
    y[...] = disv * jnp.dot(x[...], w[...], preferred_element_type=jnp.float32)
    dis[...] = disv


def _mmA(d0, d1, x, w):
    return pl.pallas_call(
        _mmA_body,
        grid=(_ACC_ROWS // _BM,),
        in_specs=[
            pl.BlockSpec((_BM, 1), lambda i: (i, 0)),
            pl.BlockSpec((_BM, 1), lambda i: (i, 0)),
            pl.BlockSpec((_BM, _D), lambda i: (i, 0)),
            pl.BlockSpec((_D, _D), lambda i: (0, 0)),
        ],
        out_specs=[
            pl.BlockSpec((_BM, _D), lambda i: (i, 0)),
            pl.BlockSpec((_BM, 1), lambda i: (i, 0)),
        ],
        out_shape=[
            jax.ShapeDtypeStruct((_ACC_ROWS, _D), jnp.float32),
            jax.ShapeDtypeStruct((_ACC_ROWS, 1), jnp.float32),
        ],
    )(d0, d1, x, w)


def _mmB_body(p, y1, dis, w, b, y2):
    pv = p[...]
    a = pv[0] + pv[1] - y1[...]
    h = jnp.maximum(dis[...] * a + b[...], 0.0)
    y2[...] = dis[...] * jnp.dot(h, w[...], preferred_element_type=jnp.float32)


def _mmB(p, y1, dis, w, b):
    return pl.pallas_call(
        _mmB_body,
        grid=(_ACC_ROWS // _BM,),
        in_specs=[
            pl.BlockSpec((_NC, _BM, _D), lambda i: (0, i, 0)),
            pl.BlockSpec((_BM, _D), lambda i: (i, 0)),
            pl.BlockSpec((_BM, 1), lambda i: (i, 0)),
            pl.BlockSpec((_D, _D), lambda i: (0, 0)),
            pl.BlockSpec((1, _D), lambda i: (0, 0)),
        ],
        out_specs=pl.BlockSpec((_BM, _D), lambda i: (i, 0)),
        out_shape=jax.ShapeDtypeStruct((_ACC_ROWS, _D), jnp.float32),
    )(p, y1, dis, w, b)


def _mmC_body(q, y2, dis, b, o):
    qv = q[...]
    o[...] = dis[...] * (qv[0] + qv[1] - y2[...]) + b[...]


def _mmC(q, y2, dis, b):
    return pl.pallas_call(
        _mmC_body,
        grid=(_ACC_ROWS // _BM,),
        in_specs=[
            pl.BlockSpec((_NC, _BM, _D), lambda i: (0, i, 0)),
            pl.BlockSpec((_BM, _D), lambda i: (i, 0)),
            pl.BlockSpec((_BM, 1), lambda i: (i, 0)),
            pl.BlockSpec((1, _D), lambda i: (0, 0)),
        ],
        out_specs=pl.BlockSpec((_BM, _D), lambda i: (i, 0)),
        out_shape=jax.ShapeDtypeStruct((_ACC_ROWS, _D), jnp.float32),
    )(q, y2, dis, b)


def kernel(x, edge_index, W1, b1, W2, b2):
    ei = edge_index.astype(jnp.int32)
    src = ei[0]
    dst = ei[1]
    npad = _EPAD - _E
    srcp = jnp.concatenate([src, jnp.zeros((npad,), jnp.int32)]).reshape(
        _NW, _NCH, _G
    )
    dstp = jnp.concatenate([dst, jnp.full((npad,), _N, jnp.int32)]).reshape(
        _NW, _NCH, _G
    )
    deg_kernel, spmm_kernel = _sc_kernels()
    dega, degb = deg_kernel(dstp)
    xp = jnp.concatenate(
        [x, jnp.zeros((_ACC_ROWS - _N, _D), jnp.float32)]
    )
    y1, dis = _mmA(dega[:, None], degb[:, None], xp, W1)
    p = spmm_kernel(y1, srcp, dstp)
    y2 = _mmB(p, y1, dis, W2, b1.reshape(1, _D))
    q = spmm_kernel(y2, srcp, dstp)
    out = _mmC(q, y2, dis, b2.reshape(1, _D))
    return out[:_N]
